# baseline (speedup 1.0000x reference)
import functools

import jax
import jax.numpy as jnp
from jax import lax
from jax.experimental import pallas as pl
from jax.experimental.pallas import tpu as pltpu
from jax.experimental.pallas import tpu_sc as plsc

NN = 10000
NP = 10240
KK = 16
NF = 64
BN = 256
GRID = NP // BN
BNP = 200
GRIDP = NN // BNP
EE = NP * KK

NW = 32
SC_PER_W = EE // NW
SC_CH = 128
SC_NCHUNK = SC_PER_W // SC_CH
GE_CH = 256
GE_NCHUNK = SC_PER_W // GE_CH
GE_GROUPS = GE_CH // 16
RBF_ROWS = EE * KK // 128
EV_ROWS = EE * 4 // 128

LOG2 = 0.6931471805599453


def _ssp(x):
    return jnp.maximum(x, 0.0) + jnp.log(1.0 + jnp.exp(-jnp.abs(x))) - LOG2


def _pre_body(feat_ref, w0_ref, w1_ref, out_ref):
    f = feat_ref[...]
    w0 = w0_ref[...]
    w1 = w1_ref[...]
    parts = [jnp.dot(f[:, :NF], w0, preferred_element_type=jnp.float32)]
    for c in range(3):
        parts.append(jnp.dot(f[:, NF * (1 + c):NF * (2 + c)], w1,
                             preferred_element_type=jnp.float32))
    out_ref[...] = jnp.concatenate(parts, axis=-1)


def _pre_call(feat, Wpre0, Wpre1):
    return pl.pallas_call(
        _pre_body,
        grid=(GRIDP,),
        in_specs=[
            pl.BlockSpec((BNP, 4 * NF), lambda i: (i, 0)),
            pl.BlockSpec((NF, NF), lambda i: (0, 0)),
            pl.BlockSpec((NF, NF), lambda i: (0, 0)),
        ],
        out_specs=pl.BlockSpec((BNP, 4 * NF), lambda i: (i, 0)),
        out_shape=jax.ShapeDtypeStruct((NN, 4 * NF), jnp.float32),
    )(feat, Wpre0, Wpre1)


def _sc_body(pre_hbm, xyzc_hbm, idx_hbm, mask_hbm,
             out_pre_hbm, out_rbf_hbm, out_ev_hbm,
             xyz_v, idx_v, mask_v, bufa_v, bufb_v, rbf_v, ev_v,
             sema, semb):
    nc = 2
    wid = lax.axis_index("s") * nc + lax.axis_index("c")
    wbase = wid * SC_PER_W
    pltpu.sync_copy(xyzc_hbm, xyz_v)
    pltpu.sync_copy(idx_hbm.at[pl.ds(wbase, SC_PER_W)], idx_v)
    pltpu.sync_copy(mask_hbm.at[pl.ds(wbase, SC_PER_W)], mask_v)

    lanes = lax.iota(jnp.int32, 16)
    zero16 = jnp.zeros((16,), jnp.int32)
    one16 = zero16 + 1
    two16 = zero16 + 2
    nbase = wbase - (wbase // NP) * NP

    def geo_chunk(ci, carry):
        cbase = ci * GE_CH

        def group(g, carry2):
            lb = cbase + g * 16
            lc = g * 16
            e_idx = idx_v[pl.ds(lb, 16)]

            def gat(w):
                return plsc.load_gather(
                    xyz_v, [lax.shift_right_logical(w, 7), w & 127])

            ew = lax.shift_left(e_idx, 2)
            sx = gat(ew)
            sy = gat(ew + 1)
            sz = gat(ew + 2)
            n_idx = (nbase + lb) + lanes
            nw = lax.shift_left(n_idx, 2)
            cx = gat(nw)
            cy = gat(nw + 1)
            cz = gat(nw + 2)
            m = mask_v[pl.ds(lb, 16)]
            rx = sx - cx
            ry = sy - cy
            rz = sz - cz
            d2 = rx * rx + ry * ry + rz * rz + 1e-12
            seed = plsc.bitcast(
                0x5F3759DF - lax.shift_right_logical(plsc.bitcast(d2, jnp.int32), 1),
                jnp.float32)
            h2 = 0.5 * d2
            x = seed * (1.5 - h2 * seed * seed)
            x = x * (1.5 - h2 * x * x)
            x = x * (1.5 - h2 * x * x)
            d = d2 * x
            t = d * 0.2
            t3 = t * t * t
            env = jnp.where(t < 1.0,
                            1.0 - 10.0 * t3 + 15.0 * t3 * t - 6.0 * t3 * t * t,
                            0.0)
            scale = env * x * m
            le4 = (lc + lanes) * 4

            def scat(ref, w, val):
                plsc.store_scatter(
                    ref, [lax.shift_right_logical(w, 7), w & 127], val)

            scat(ev_v, le4, ry * x)
            scat(ev_v, le4 + 1, rz * x)
            scat(ev_v, le4 + 2, rx * x)
            tc = jnp.minimum(t, 1.0)
            x2 = tc * tc
            s1 = tc * (1.0 + x2 * (-1.0 / 6.0 + x2 * (1.0 / 120.0
                      + x2 * (-1.0 / 5040.0 + x2 * (1.0 / 362880.0)))))
            c1 = 1.0 + x2 * (-0.5 + x2 * (1.0 / 24.0 + x2 * (-1.0 / 720.0
                      + x2 * (1.0 / 40320.0 + x2 * (-1.0 / 3628800.0)))))
            twoc = 2.0 * c1
            le16 = (lc + lanes) * 16
            scat(rbf_v, le16, jnp.zeros((16,), jnp.float32))
            sp = jnp.zeros((16,), jnp.float32)
            sc = s1
            scat(rbf_v, le16 + 1, sc * scale)
            for f in range(2, KK):
                sp, sc = sc, twoc * sc - sp
                scat(rbf_v, le16 + f, sc * scale)
            return carry2

        lax.fori_loop(0, GE_GROUPS, group, 0)
        pltpu.sync_copy(rbf_v,
                        out_rbf_hbm.at[pl.ds(wid * (SC_PER_W * KK // 128)
                                             + ci * (GE_CH * KK // 128),
                                             GE_CH * KK // 128)])
        pltpu.sync_copy(ev_v,
                        out_ev_hbm.at[pl.ds(wid * (SC_PER_W * 4 // 128)
                                            + ci * (GE_CH * 4 // 128),
                                            GE_CH * 4 // 128)])
        return carry

    lax.fori_loop(0, GE_NCHUNK, geo_chunk, 0)

    def gather(j, buf, sem):
        idx_c = idx_v.at[pl.ds(j * SC_CH, SC_CH)]
        return pltpu.async_copy(pre_hbm.at[idx_c], buf, sem)

    def writeback(j, buf):
        pltpu.sync_copy(buf, out_pre_hbm.at[pl.ds(wbase + j * SC_CH, SC_CH)])

    cpa = gather(0, bufa_v, sema)

    def pair(i, carry):
        j0 = 2 * i
        cpb = gather(j0 + 1, bufb_v, semb)
        pltpu.make_async_copy(pre_hbm.at[idx_v.at[pl.ds(j0 * SC_CH, SC_CH)]],
                              bufa_v, sema).wait()
        writeback(j0, bufa_v)
        jn = jnp.minimum(j0 + 2, SC_NCHUNK - 1)
        cpa2 = gather(jn, bufa_v, sema)
        pltpu.make_async_copy(
            pre_hbm.at[idx_v.at[pl.ds((j0 + 1) * SC_CH, SC_CH)]],
            bufb_v, semb).wait()
        writeback(j0 + 1, bufb_v)
        return carry

    lax.fori_loop(0, SC_NCHUNK // 2, pair, 0)
    pltpu.make_async_copy(pre_hbm.at[idx_v.at[pl.ds(0, SC_CH)]],
                          bufa_v, sema).wait()


def _sc_call(pre, xyzc, idx1d, mask1d):
    mesh = plsc.VectorSubcoreMesh(core_axis_name="c", subcore_axis_name="s")
    fn = pl.kernel(
        _sc_body,
        mesh=mesh,
        out_type=[
            jax.ShapeDtypeStruct((EE, 4 * NF), jnp.float32),
            jax.ShapeDtypeStruct((RBF_ROWS, 128), jnp.float32),
            jax.ShapeDtypeStruct((EV_ROWS, 128), jnp.float32),
        ],
        scratch_types=[
            pltpu.VMEM((NP * 4 // 128, 128), jnp.float32),
            pltpu.VMEM((SC_PER_W,), jnp.int32),
            pltpu.VMEM((SC_PER_W,), jnp.float32),
            pltpu.VMEM((SC_CH, 4 * NF), jnp.float32),
            pltpu.VMEM((SC_CH, 4 * NF), jnp.float32),
            pltpu.VMEM((GE_CH * KK // 128, 128), jnp.float32),
            pltpu.VMEM((GE_CH * 4 // 128, 128), jnp.float32),
            pltpu.SemaphoreType.DMA,
            pltpu.SemaphoreType.DMA,
        ],
        compiler_params=pltpu.CompilerParams(needs_layout_passes=False),
    )
    return fn(pre, xyzc, idx1d, mask1d)


def _edge_body(pre_e_ref, rbf_ref, ev_ref, feat_ref,
               wf1_ref, wf2_ref, wc0_ref, wc1_ref, wo0_ref, wo1_ref,
               out_ref):
    BE = KK * BN
    pe = pre_e_ref[...].reshape(BE, 4 * NF)
    rbf = rbf_ref[...].reshape(BE, KK)
    ev4 = ev_ref[...].reshape(BE, 4)

    fr = jnp.dot(_ssp(jnp.dot(rbf, wf1_ref[...],
                              preferred_element_type=jnp.float32)),
                 wf2_ref[...], preferred_element_type=jnp.float32)

    s1 = pe[:, :NF]
    v1 = [pe[:, NF * (1 + c):NF * (2 + c)] for c in range(3)]
    ev = [jnp.broadcast_to(ev4[:, c:c + 1], (BE, NF)) for c in range(3)]

    p0 = s1 * fr
    p = [v1[c] * fr for c in range(3)]
    ch = [p0,
          p[0] * ev[0] + p[1] * ev[1] + p[2] * ev[2],
          ev[0] * p0, ev[1] * p0, ev[2] * p0,
          p[0], p[1], p[2],
          p[1] * ev[2] - p[2] * ev[1],
          p[2] * ev[0] - p[0] * ev[2],
          p[0] * ev[1] - p[1] * ev[0]]

    def ksum(x):
        acc = x[0:BN]
        for k in range(1, KK):
            acc = acc + x[k * BN:(k + 1) * BN]
        return acc

    acc_sa = ksum(ch[0])
    acc_sb = ksum(ch[1])
    acc01 = [ksum(ch[2 + c]) for c in range(3)]
    acc10 = [ksum(ch[5 + c]) for c in range(3)]
    acc11 = [ksum(ch[8 + c]) for c in range(3)]

    wc0 = wc0_ref[...]
    wc1 = wc1_ref[...]
    conv_s = jnp.dot(jnp.concatenate([acc_sa, acc_sb], axis=-1), wc0,
                     preferred_element_type=jnp.float32)
    conv_v = [jnp.dot(jnp.concatenate([acc01[c], acc10[c], acc11[c]], axis=-1),
                      wc1, preferred_element_type=jnp.float32)
              for c in range(3)]

    inv_norm = jnp.sqrt(conv_v[0] * conv_v[0] + conv_v[1] * conv_v[1]
                        + conv_v[2] * conv_v[2] + 1e-12)
    g0 = _ssp(conv_s)
    g1 = _ssp(inv_norm)
    wo0 = wo0_ref[...]
    wo1 = wo1_ref[...]
    so = jnp.dot(conv_s * g0, wo0, preferred_element_type=jnp.float32)
    vo = [jnp.dot(conv_v[c] * g1, wo1, preferred_element_type=jnp.float32)
          for c in range(3)]
    out_ref[...] = feat_ref[...] + jnp.concatenate([so] + vo, axis=-1)


def _edge_call(pre_e, rbf_e, ev_e, feat_pad,
               W_f1, W_f2, Wc0, Wc1, Wo0, Wo1):
    return pl.pallas_call(
        _edge_body,
        grid=(GRID,),
        in_specs=[
            pl.BlockSpec((KK, BN, 4 * NF), lambda i: (0, i, 0)),
            pl.BlockSpec((KK, BN, KK), lambda i: (0, i, 0)),
            pl.BlockSpec((KK, BN, 4), lambda i: (0, i, 0)),
            pl.BlockSpec((BN, 4 * NF), lambda i: (i, 0)),
            pl.BlockSpec((KK, NF), lambda i: (0, 0)),
            pl.BlockSpec((NF, NF), lambda i: (0, 0)),
            pl.BlockSpec((2 * NF, NF), lambda i: (0, 0)),
            pl.BlockSpec((3 * NF, NF), lambda i: (0, 0)),
            pl.BlockSpec((NF, NF), lambda i: (0, 0)),
            pl.BlockSpec((NF, NF), lambda i: (0, 0)),
        ],
        out_specs=pl.BlockSpec((BN, 4 * NF), lambda i: (i, 0)),
        out_shape=jax.ShapeDtypeStruct((NP, 4 * NF), jnp.float32),
    )(pre_e, rbf_e, ev_e, feat_pad, W_f1, W_f2, Wc0, Wc1, Wo0, Wo1)


def kernel(xyz, feat, edge_mask, W_f1, W_f2, Wpre0, Wpre1, Wc0, Wc1,
           Wo0, Wo1, nbr_idx):
    xyzc = jnp.zeros((NP, 4), jnp.float32).at[:NN, 0:3].set(xyz)
    xyzc = xyzc.reshape(NP * 4 // 128, 128)
    idx1d = jnp.zeros((KK, NP), jnp.int32).at[:, :NN].set(
        nbr_idx.T.astype(jnp.int32)).reshape(EE)
    mask1d = jnp.zeros((KK, NP), jnp.float32).at[:, :NN].set(
        edge_mask.T).reshape(EE)
    feat_pad = jnp.zeros((NP, 4 * NF), jnp.float32).at[:NN].set(feat)

    pre = _pre_call(feat, Wpre0, Wpre1)
    pre_e, rbf_e, ev_e = _sc_call(pre, xyzc, idx1d, mask1d)
    out = _edge_call(pre_e.reshape(KK, NP, 4 * NF),
                     rbf_e.reshape(KK, NP, KK),
                     ev_e.reshape(KK, NP, 4),
                     feat_pad,
                     W_f1, W_f2, Wc0, Wc1, Wo0, Wo1)
    return out[:NN]


# speedup vs baseline: 1.1091x; 1.1091x over previous
import functools

import jax
import jax.numpy as jnp
from jax import lax
from jax.experimental import pallas as pl
from jax.experimental.pallas import tpu as pltpu
from jax.experimental.pallas import tpu_sc as plsc

NN = 10000
NP = 10240
KK = 16
NF = 64
BN = 256
GRID = NP // BN
BNP = 200
GRIDP = NN // BNP
EE = NP * KK

NW = 32
SC_PER_W = EE // NW
SC_CH = 40
SC_NCHUNK = SC_PER_W // SC_CH
GE_CH = 256
GE_NCHUNK = SC_PER_W // GE_CH
GE_GROUPS = GE_CH // 16
RBF_ROWS = EE * KK // 128
EV_ROWS = EE * 4 // 128

LOG2 = 0.6931471805599453


def _ssp(x):
    return jnp.maximum(x, 0.0) + jnp.log(1.0 + jnp.exp(-jnp.abs(x))) - LOG2


def _pre_body(feat_ref, w0_ref, w1_ref, out_ref):
    f = feat_ref[...]
    w0 = w0_ref[...]
    w1 = w1_ref[...]
    parts = [jnp.dot(f[:, :NF], w0, preferred_element_type=jnp.float32)]
    for c in range(3):
        parts.append(jnp.dot(f[:, NF * (1 + c):NF * (2 + c)], w1,
                             preferred_element_type=jnp.float32))
    pref = jnp.concatenate(parts, axis=-1)
    lob = lax.bitcast_convert_type(
        pref[:, :2 * NF].astype(jnp.bfloat16).astype(jnp.float32), jnp.uint32)
    hib = lax.bitcast_convert_type(
        pref[:, 2 * NF:].astype(jnp.bfloat16).astype(jnp.float32), jnp.uint32)
    out_ref[...] = lax.bitcast_convert_type(
        (lob >> 16) | hib, jnp.float32)


def _pre_call(feat, Wpre0, Wpre1):
    return pl.pallas_call(
        _pre_body,
        grid=(GRIDP,),
        in_specs=[
            pl.BlockSpec((BNP, 4 * NF), lambda i: (i, 0)),
            pl.BlockSpec((NF, NF), lambda i: (0, 0)),
            pl.BlockSpec((NF, NF), lambda i: (0, 0)),
        ],
        out_specs=pl.BlockSpec((BNP, 2 * NF), lambda i: (i, 0)),
        out_shape=jax.ShapeDtypeStruct((NN, 2 * NF), jnp.float32),
    )(feat, Wpre0, Wpre1)


def _sc_body(pre_hbm, xyzc_hbm, idx_hbm, mask_hbm,
             out_pre_hbm, out_rbf_hbm, out_ev_hbm,
             xyz_v, idx_v, mask_v, bufa_v, bufb_v, rbf_v, ev_v,
             sema, semb):
    nc = 2
    wid = lax.axis_index("s") * nc + lax.axis_index("c")
    wbase = wid * SC_PER_W
    pltpu.sync_copy(xyzc_hbm, xyz_v)
    pltpu.sync_copy(idx_hbm.at[pl.ds(wbase, SC_PER_W)], idx_v)
    pltpu.sync_copy(mask_hbm.at[pl.ds(wbase, SC_PER_W)], mask_v)

    lanes = lax.iota(jnp.int32, 16)
    zero16 = jnp.zeros((16,), jnp.int32)
    one16 = zero16 + 1
    two16 = zero16 + 2
    nbase = wbase - (wbase // NP) * NP

    def geo_chunk(ci, carry):
        cbase = ci * GE_CH

        def group(g, carry2):
            lb = cbase + g * 16
            lc = g * 16
            e_idx = idx_v[pl.ds(lb, 16)]

            def gat(w):
                return plsc.load_gather(
                    xyz_v, [lax.shift_right_logical(w, 7), w & 127])

            ew = lax.shift_left(e_idx, 2)
            sx = gat(ew)
            sy = gat(ew + 1)
            sz = gat(ew + 2)
            n_idx = (nbase + lb) + lanes
            nw = lax.shift_left(n_idx, 2)
            cx = gat(nw)
            cy = gat(nw + 1)
            cz = gat(nw + 2)
            m = mask_v[pl.ds(lb, 16)]
            rx = sx - cx
            ry = sy - cy
            rz = sz - cz
            d2 = rx * rx + ry * ry + rz * rz + 1e-12
            seed = plsc.bitcast(
                0x5F3759DF - lax.shift_right_logical(plsc.bitcast(d2, jnp.int32), 1),
                jnp.float32)
            h2 = 0.5 * d2
            x = seed * (1.5 - h2 * seed * seed)
            x = x * (1.5 - h2 * x * x)
            x = x * (1.5 - h2 * x * x)
            d = d2 * x
            t = d * 0.2
            t3 = t * t * t
            env = jnp.where(t < 1.0,
                            1.0 - 10.0 * t3 + 15.0 * t3 * t - 6.0 * t3 * t * t,
                            0.0)
            scale = env * x * m
            le4 = (lc + lanes) * 4

            def scat(ref, w, val):
                plsc.store_scatter(
                    ref, [lax.shift_right_logical(w, 7), w & 127], val)

            scat(ev_v, le4, ry * x)
            scat(ev_v, le4 + 1, rz * x)
            scat(ev_v, le4 + 2, rx * x)
            tc = jnp.minimum(t, 1.0)
            x2 = tc * tc
            s1 = tc * (1.0 + x2 * (-1.0 / 6.0 + x2 * (1.0 / 120.0
                      + x2 * (-1.0 / 5040.0 + x2 * (1.0 / 362880.0)))))
            c1 = 1.0 + x2 * (-0.5 + x2 * (1.0 / 24.0 + x2 * (-1.0 / 720.0
                      + x2 * (1.0 / 40320.0 + x2 * (-1.0 / 3628800.0)))))
            twoc = 2.0 * c1
            le16 = (lc + lanes) * 16
            scat(rbf_v, le16, jnp.zeros((16,), jnp.float32))
            sp = jnp.zeros((16,), jnp.float32)
            sc = s1
            scat(rbf_v, le16 + 1, sc * scale)
            for f in range(2, KK):
                sp, sc = sc, twoc * sc - sp
                scat(rbf_v, le16 + f, sc * scale)
            return carry2

        lax.fori_loop(0, GE_GROUPS, group, 0)
        pltpu.sync_copy(rbf_v,
                        out_rbf_hbm.at[pl.ds(wid * (SC_PER_W * KK // 128)
                                             + ci * (GE_CH * KK // 128),
                                             GE_CH * KK // 128)])
        pltpu.sync_copy(ev_v,
                        out_ev_hbm.at[pl.ds(wid * (SC_PER_W * 4 // 128)
                                            + ci * (GE_CH * 4 // 128),
                                            GE_CH * 4 // 128)])
        return carry

    lax.fori_loop(0, GE_NCHUNK, geo_chunk, 0)

    def gather(j, buf, sem):
        idx_c = idx_v.at[pl.ds(j * SC_CH, SC_CH)]
        return pltpu.async_copy(pre_hbm.at[idx_c], buf, sem)

    def writeback(j, buf):
        pltpu.sync_copy(buf, out_pre_hbm.at[pl.ds(wbase + j * SC_CH, SC_CH)])

    cpa = gather(0, bufa_v, sema)

    def pair(i, carry):
        j0 = 2 * i
        cpb = gather(j0 + 1, bufb_v, semb)
        pltpu.make_async_copy(pre_hbm.at[idx_v.at[pl.ds(j0 * SC_CH, SC_CH)]],
                              bufa_v, sema).wait()
        writeback(j0, bufa_v)
        jn = jnp.minimum(j0 + 2, SC_NCHUNK - 1)
        cpa2 = gather(jn, bufa_v, sema)
        pltpu.make_async_copy(
            pre_hbm.at[idx_v.at[pl.ds((j0 + 1) * SC_CH, SC_CH)]],
            bufb_v, semb).wait()
        writeback(j0 + 1, bufb_v)
        return carry

    lax.fori_loop(0, SC_NCHUNK // 2, pair, 0)
    pltpu.make_async_copy(pre_hbm.at[idx_v.at[pl.ds(0, SC_CH)]],
                          bufa_v, sema).wait()


def _sc_call(pre, xyzc, idx1d, mask1d):
    mesh = plsc.VectorSubcoreMesh(core_axis_name="c", subcore_axis_name="s")
    fn = pl.kernel(
        _sc_body,
        mesh=mesh,
        out_type=[
            jax.ShapeDtypeStruct((EE, 2 * NF), jnp.float32),
            jax.ShapeDtypeStruct((RBF_ROWS, 128), jnp.float32),
            jax.ShapeDtypeStruct((EV_ROWS, 128), jnp.float32),
        ],
        scratch_types=[
            pltpu.VMEM((NP * 4 // 128, 128), jnp.float32),
            pltpu.VMEM((SC_PER_W,), jnp.int32),
            pltpu.VMEM((SC_PER_W,), jnp.float32),
            pltpu.VMEM((SC_CH, 2 * NF), jnp.float32),
            pltpu.VMEM((SC_CH, 2 * NF), jnp.float32),
            pltpu.VMEM((GE_CH * KK // 128, 128), jnp.float32),
            pltpu.VMEM((GE_CH * 4 // 128, 128), jnp.float32),
            pltpu.SemaphoreType.DMA,
            pltpu.SemaphoreType.DMA,
        ],
        compiler_params=pltpu.CompilerParams(needs_layout_passes=False),
    )
    return fn(pre, xyzc, idx1d, mask1d)


def _edge_body(pre_e_ref, rbf_ref, ev_ref, feat_ref,
               wf1_ref, wf2_ref, wc0_ref, wc1_ref, wo0_ref, wo1_ref,
               out_ref):
    BE = KK * BN
    pw = lax.bitcast_convert_type(pre_e_ref[...].reshape(BE, 2 * NF),
                                  jnp.uint32)
    flo = lax.bitcast_convert_type(pw << 16, jnp.float32)
    fhi = lax.bitcast_convert_type(pw & jnp.uint32(0xFFFF0000),
                                   jnp.float32)
    rbf = rbf_ref[...].reshape(BE, KK)
    ev4 = ev_ref[...].reshape(BE, 4)

    fr = jnp.dot(_ssp(jnp.dot(rbf, wf1_ref[...],
                              preferred_element_type=jnp.float32)),
                 wf2_ref[...], preferred_element_type=jnp.float32)

    s1 = flo[:, :NF]
    v1 = [flo[:, NF:], fhi[:, :NF], fhi[:, NF:]]
    ev = [jnp.broadcast_to(ev4[:, c:c + 1], (BE, NF)) for c in range(3)]

    p0 = s1 * fr
    p = [v1[c] * fr for c in range(3)]
    ch = [p0,
          p[0] * ev[0] + p[1] * ev[1] + p[2] * ev[2],
          ev[0] * p0, ev[1] * p0, ev[2] * p0,
          p[0], p[1], p[2],
          p[1] * ev[2] - p[2] * ev[1],
          p[2] * ev[0] - p[0] * ev[2],
          p[0] * ev[1] - p[1] * ev[0]]

    def ksum(x):
        acc = x[0:BN]
        for k in range(1, KK):
            acc = acc + x[k * BN:(k + 1) * BN]
        return acc

    acc_sa = ksum(ch[0])
    acc_sb = ksum(ch[1])
    acc01 = [ksum(ch[2 + c]) for c in range(3)]
    acc10 = [ksum(ch[5 + c]) for c in range(3)]
    acc11 = [ksum(ch[8 + c]) for c in range(3)]

    wc0 = wc0_ref[...]
    wc1 = wc1_ref[...]
    conv_s = jnp.dot(jnp.concatenate([acc_sa, acc_sb], axis=-1), wc0,
                     preferred_element_type=jnp.float32)
    conv_v = [jnp.dot(jnp.concatenate([acc01[c], acc10[c], acc11[c]], axis=-1),
                      wc1, preferred_element_type=jnp.float32)
              for c in range(3)]

    inv_norm = jnp.sqrt(conv_v[0] * conv_v[0] + conv_v[1] * conv_v[1]
                        + conv_v[2] * conv_v[2] + 1e-12)
    g0 = _ssp(conv_s)
    g1 = _ssp(inv_norm)
    wo0 = wo0_ref[...]
    wo1 = wo1_ref[...]
    so = jnp.dot(conv_s * g0, wo0, preferred_element_type=jnp.float32)
    vo = [jnp.dot(conv_v[c] * g1, wo1, preferred_element_type=jnp.float32)
          for c in range(3)]
    out_ref[...] = feat_ref[...] + jnp.concatenate([so] + vo, axis=-1)


def _edge_call(pre_e, rbf_e, ev_e, feat_pad,
               W_f1, W_f2, Wc0, Wc1, Wo0, Wo1):
    return pl.pallas_call(
        _edge_body,
        grid=(GRID,),
        in_specs=[
            pl.BlockSpec((KK, BN, 2 * NF), lambda i: (0, i, 0)),
            pl.BlockSpec((KK, BN, KK), lambda i: (0, i, 0)),
            pl.BlockSpec((KK, BN, 4), lambda i: (0, i, 0)),
            pl.BlockSpec((BN, 4 * NF), lambda i: (i, 0)),
            pl.BlockSpec((KK, NF), lambda i: (0, 0)),
            pl.BlockSpec((NF, NF), lambda i: (0, 0)),
            pl.BlockSpec((2 * NF, NF), lambda i: (0, 0)),
            pl.BlockSpec((3 * NF, NF), lambda i: (0, 0)),
            pl.BlockSpec((NF, NF), lambda i: (0, 0)),
            pl.BlockSpec((NF, NF), lambda i: (0, 0)),
        ],
        out_specs=pl.BlockSpec((BN, 4 * NF), lambda i: (i, 0)),
        out_shape=jax.ShapeDtypeStruct((NP, 4 * NF), jnp.float32),
    )(pre_e, rbf_e, ev_e, feat_pad, W_f1, W_f2, Wc0, Wc1, Wo0, Wo1)


def kernel(xyz, feat, edge_mask, W_f1, W_f2, Wpre0, Wpre1, Wc0, Wc1,
           Wo0, Wo1, nbr_idx):
    xyzc = jnp.zeros((NP, 4), jnp.float32).at[:NN, 0:3].set(xyz)
    xyzc = xyzc.reshape(NP * 4 // 128, 128)
    idx1d = jnp.zeros((KK, NP), jnp.int32).at[:, :NN].set(
        nbr_idx.T.astype(jnp.int32)).reshape(EE)
    mask1d = jnp.zeros((KK, NP), jnp.float32).at[:, :NN].set(
        edge_mask.T).reshape(EE)
    feat_pad = jnp.zeros((NP, 4 * NF), jnp.float32).at[:NN].set(feat)

    pre = _pre_call(feat, Wpre0, Wpre1)
    pre_e, rbf_e, ev_e = _sc_call(pre, xyzc, idx1d, mask1d)
    out = _edge_call(pre_e.reshape(KK, NP, 2 * NF),
                     rbf_e.reshape(KK, NP, KK),
                     ev_e.reshape(KK, NP, 4),
                     feat_pad,
                     W_f1, W_f2, Wc0, Wc1, Wo0, Wo1)
    return out[:NN]
